# Initial kernel scaffold; baseline (speedup 1.0000x reference)
import jax
import jax.numpy as jnp
from jax.experimental import pallas as pl


def kernel(u, in_proj_w, conv_w, conv_b, dt_bias, A_log, D_param, norm_w, out_proj_w):
    raise NotImplementedError("write your pallas kernel here")


# speedup vs baseline: 56.7505x; 56.7505x over previous
import jax
import jax.numpy as jnp
from jax.experimental import pallas as pl
from jax.experimental.pallas import tpu as pltpu

D_MODEL = 1024
D_STATE = 64
D_CONV = 4
HEADDIM = 128
NHEADS = 16
D_INNER = 2048
CONV_DIM = 2176
BATCH = 2
SEQLEN = 2048
Q = 256
RMS_EPS = 1e-5
NEG_BIG = -1e30


def _softplus(x):
    return jnp.maximum(x, 0.0) + jnp.log1p(jnp.exp(-jnp.abs(x)))


def _mamba_kernel(u_ref, wz_ref, wxbc_ref, wdt_ref, wdt_o_ref, cw_ref, cb_ref,
                  dtb_row_ref, dtb_col_ref, a_row_ref, a_col_ref, d_row_ref,
                  nw_ref, wo_ref, o_ref, tail_ref, state_ref):
    t = pl.program_id(1)

    @pl.when(t == 0)
    def _():
        tail_ref[...] = jnp.zeros_like(tail_ref)
        state_ref[...] = jnp.zeros_like(state_ref)

    uv = u_ref[0]

    z = jnp.dot(uv, wz_ref[...], preferred_element_type=jnp.float32)
    xbc = jnp.dot(uv, wxbc_ref[...], preferred_element_type=jnp.float32)
    dtr = jnp.dot(uv, wdt_ref[...], preferred_element_type=jnp.float32)
    dtr_t = jax.lax.dot_general(wdt_o_ref[...], uv, (((1,), (1,)), ((), ())),
                                preferred_element_type=jnp.float32)

    full = jnp.concatenate([tail_ref[0:D_CONV - 1], xbc], axis=0)
    tail_ref[0:D_CONV - 1] = xbc[Q - (D_CONV - 1):Q]
    conv = (cw_ref[0:1] * full[0:Q] + cw_ref[1:2] * full[1:Q + 1]
            + cw_ref[2:3] * full[2:Q + 2] + cw_ref[3:4] * full[3:Q + 3]
            + cb_ref[...])
    xbc_a = conv * jax.nn.sigmoid(conv)

    x_full = xbc_a[:, :D_INNER]
    bmat = xbc_a[:, D_INNER:D_INNER + D_STATE]
    cmat = xbc_a[:, D_INNER + D_STATE:]

    dt = _softplus(dtr + dtb_row_ref[...])
    dt_t = _softplus(dtr_t + dtb_col_ref[...])
    a = dt * a_row_ref[...]
    a_t = dt_t * a_col_ref[...]

    ri = jax.lax.broadcasted_iota(jnp.int32, (Q, Q), 0)
    ci = jax.lax.broadcasted_iota(jnp.int32, (Q, Q), 1)
    causal = ci <= ri
    trif = jnp.where(causal, 1.0, 0.0)
    negmask = jnp.where(causal, 0.0, NEG_BIG)

    s = jnp.dot(trif, a, preferred_element_type=jnp.float32)
    s_t = jax.lax.dot_general(a_t, trif, (((1,), (1,)), ((), ())),
                              preferred_element_type=jnp.float32)

    es16 = jnp.exp(s)
    es_last = es16[Q - 1:Q, :]
    wd16 = jnp.exp(s[Q - 1:Q, :] - s) * dt

    he = jax.lax.broadcasted_iota(jnp.int32, (16, D_INNER), 0)
    le = jax.lax.broadcasted_iota(jnp.int32, (16, D_INNER), 1)
    expand = jnp.where(he == (le >> 7), 1.0, 0.0)
    dt_exp = jnp.dot(dt, expand, preferred_element_type=jnp.float32)
    wd_exp = jnp.dot(wd16, expand, preferred_element_type=jnp.float32)
    es_exp = jnp.dot(es16, expand, preferred_element_type=jnp.float32)

    dtx = x_full * dt_exp
    xw = x_full * wd_exp

    g = jax.lax.dot_general(cmat, bmat, (((1,), (1,)), ((), ())),
                            preferred_element_type=jnp.float32)

    hi16 = jax.lax.broadcasted_iota(jnp.int32, (16, Q), 0)
    ys = []
    for h in range(NHEADS):
        lo, hihd = h * HEADDIM, (h + 1) * HEADDIM
        w1h = jnp.where(hi16 == h, 1.0, 0.0)
        col_s = jnp.dot(s, w1h, preferred_element_type=jnp.float32)
        row_s = s_t[h:h + 1, :]
        m = jnp.exp(col_s - row_s + negmask)
        p = g * m
        s_h = state_ref[h]
        y_h = (jnp.dot(p, dtx[:, lo:hihd], preferred_element_type=jnp.float32)
               + es_exp[:, lo:hihd]
               * jnp.dot(cmat, s_h, preferred_element_type=jnp.float32))
        contrib = jax.lax.dot_general(bmat, xw[:, lo:hihd],
                                      (((0,), (0,)), ((), ())),
                                      preferred_element_type=jnp.float32)
        state_ref[h] = s_h * es_last[0:1, h:h + 1] + contrib
        ys.append(y_h)

    y = jnp.concatenate(ys, axis=1) + x_full * d_row_ref[...]

    yg = y * (z * jax.nn.sigmoid(z))
    ms = jnp.mean(yg * yg, axis=-1, keepdims=True)
    yn = yg * jax.lax.rsqrt(ms + RMS_EPS) * nw_ref[...]

    o_ref[0] = jnp.dot(yn, wo_ref[...], preferred_element_type=jnp.float32)


def kernel(u, in_proj_w, conv_w, conv_b, dt_bias, A_log, D_param, norm_w, out_proj_w):
    f32 = jnp.float32
    wz_t = in_proj_w[:D_INNER].T
    wxbc_t = in_proj_w[D_INNER:D_INNER + CONV_DIM].T
    wdt_o = in_proj_w[D_INNER + CONV_DIM:]
    wdt_t = wdt_o.T
    cw4 = conv_w[:, 0, :].T
    cb = conv_b.reshape(1, CONV_DIM)
    dtb_row = dt_bias.reshape(1, NHEADS)
    dtb_col = dt_bias.reshape(NHEADS, 1)
    a_row = (-jnp.exp(A_log)).reshape(1, NHEADS)
    a_col = a_row.reshape(NHEADS, 1)
    d_row = jnp.repeat(D_param, HEADDIM).reshape(1, D_INNER)
    nw = norm_w.reshape(1, D_INNER)
    wo_t = out_proj_w.T

    nt = SEQLEN // Q
    grid = (BATCH, nt)
    full = lambda shp: pl.BlockSpec(shp, lambda b, t: (0,) * len(shp))
    out = pl.pallas_call(
        _mamba_kernel,
        grid=grid,
        in_specs=[
            pl.BlockSpec((1, Q, D_MODEL), lambda b, t: (b, t, 0)),
            full((D_MODEL, D_INNER)),
            full((D_MODEL, CONV_DIM)),
            full((D_MODEL, NHEADS)),
            full((NHEADS, D_MODEL)),
            full((D_CONV, CONV_DIM)),
            full((1, CONV_DIM)),
            full((1, NHEADS)),
            full((NHEADS, 1)),
            full((1, NHEADS)),
            full((NHEADS, 1)),
            full((1, D_INNER)),
            full((1, D_INNER)),
            full((D_INNER, D_MODEL)),
        ],
        out_specs=pl.BlockSpec((1, Q, D_MODEL), lambda b, t: (b, t, 0)),
        out_shape=jax.ShapeDtypeStruct((BATCH, SEQLEN, D_MODEL), f32),
        scratch_shapes=[
            pltpu.VMEM((8, CONV_DIM), f32),
            pltpu.VMEM((NHEADS, D_STATE, HEADDIM), f32),
        ],
        compiler_params=pltpu.CompilerParams(
            dimension_semantics=("parallel", "arbitrary"),
            vmem_limit_bytes=100 * 1024 * 1024,
        ),
    )(u.astype(f32), wz_t, wxbc_t, wdt_t, wdt_o, cw4, cb, dtb_row, dtb_col,
      a_row, a_col, d_row, nw, wo_t)
    return out


# speedup vs baseline: 64.3890x; 1.1346x over previous
import jax
import jax.numpy as jnp
from jax.experimental import pallas as pl
from jax.experimental.pallas import tpu as pltpu

D_MODEL = 1024
D_STATE = 64
D_CONV = 4
HEADDIM = 128
NHEADS = 16
D_INNER = 2048
CONV_DIM = 2176
BATCH = 2
SEQLEN = 2048
Q = 256
RMS_EPS = 1e-5
NEG_BIG = -1e30


def _softplus(x):
    return jnp.maximum(x, 0.0) + jnp.log1p(jnp.exp(-jnp.abs(x)))


def _mamba_kernel(u_ref, wz_ref, wxbc_ref, wdt_ref, wdt_o_ref, cw_ref, cb_ref,
                  dtb_row_ref, dtb_col_ref, a_row_ref, a_col_ref, d_row_ref,
                  nw_ref, wo_ref, o_ref, tail_ref, state_ref):
    t = pl.program_id(1)

    @pl.when(t == 0)
    def _():
        tail_ref[...] = jnp.zeros_like(tail_ref)
        state_ref[...] = jnp.zeros_like(state_ref)

    uv = u_ref[0]
    uvb = uv.astype(jnp.bfloat16)

    z = jnp.dot(uvb, wz_ref[...], preferred_element_type=jnp.float32)
    xbc = jnp.dot(uvb, wxbc_ref[...], preferred_element_type=jnp.float32)
    dtr = jnp.dot(uv, wdt_ref[...], preferred_element_type=jnp.float32)
    dtr_t = jax.lax.dot_general(wdt_o_ref[...], uv, (((1,), (1,)), ((), ())),
                                preferred_element_type=jnp.float32)

    full = jnp.concatenate([tail_ref[0:D_CONV - 1], xbc], axis=0)
    tail_ref[0:D_CONV - 1] = xbc[Q - (D_CONV - 1):Q]
    conv = (cw_ref[0:1] * full[0:Q] + cw_ref[1:2] * full[1:Q + 1]
            + cw_ref[2:3] * full[2:Q + 2] + cw_ref[3:4] * full[3:Q + 3]
            + cb_ref[...])
    xbc_a = conv * jax.nn.sigmoid(conv)

    x_full = xbc_a[:, :D_INNER]
    bmat = xbc_a[:, D_INNER:D_INNER + D_STATE]
    cmat = xbc_a[:, D_INNER + D_STATE:]

    dt = _softplus(dtr + dtb_row_ref[...])
    dt_t = _softplus(dtr_t + dtb_col_ref[...])
    a = dt * a_row_ref[...]
    a_t = dt_t * a_col_ref[...]

    ri = jax.lax.broadcasted_iota(jnp.int32, (Q, Q), 0)
    ci = jax.lax.broadcasted_iota(jnp.int32, (Q, Q), 1)
    causal = ci <= ri
    trif = jnp.where(causal, 1.0, 0.0)
    negmask = jnp.where(causal, 0.0, NEG_BIG)

    s = jnp.dot(trif, a, preferred_element_type=jnp.float32)
    s_t = jax.lax.dot_general(a_t, trif, (((1,), (1,)), ((), ())),
                              preferred_element_type=jnp.float32)

    es16 = jnp.exp(s)
    es_last = es16[Q - 1:Q, :]
    wd16 = jnp.exp(s[Q - 1:Q, :] - s) * dt

    he = jax.lax.broadcasted_iota(jnp.int32, (16, D_INNER), 0)
    le = jax.lax.broadcasted_iota(jnp.int32, (16, D_INNER), 1)
    expand = jnp.where(he == (le >> 7), 1.0, 0.0)
    dt_exp = jnp.dot(dt, expand, preferred_element_type=jnp.float32)
    wd_exp = jnp.dot(wd16, expand, preferred_element_type=jnp.float32)
    es_exp = jnp.dot(es16, expand, preferred_element_type=jnp.float32)

    dtx = x_full * dt_exp
    xw = x_full * wd_exp

    bmat_b = bmat.astype(jnp.bfloat16)
    cmat_b = cmat.astype(jnp.bfloat16)
    dtx_b = dtx.astype(jnp.bfloat16)
    xw_b = xw.astype(jnp.bfloat16)
    g = jax.lax.dot_general(cmat_b, bmat_b, (((1,), (1,)), ((), ())),
                            preferred_element_type=jnp.float32)

    hi16 = jax.lax.broadcasted_iota(jnp.int32, (16, Q), 0)
    ys = []
    for h in range(NHEADS):
        lo, hihd = h * HEADDIM, (h + 1) * HEADDIM
        w1h = jnp.where(hi16 == h, 1.0, 0.0)
        col_s = jnp.dot(s, w1h, preferred_element_type=jnp.float32)
        row_s = s_t[h:h + 1, :]
        m = jnp.exp(col_s - row_s + negmask)
        p = (g * m).astype(jnp.bfloat16)
        s_h = state_ref[h]
        y_h = (jnp.dot(p, dtx_b[:, lo:hihd], preferred_element_type=jnp.float32)
               + es_exp[:, lo:hihd]
               * jnp.dot(cmat_b, s_h.astype(jnp.bfloat16),
                         preferred_element_type=jnp.float32))
        contrib = jax.lax.dot_general(bmat_b, xw_b[:, lo:hihd],
                                      (((0,), (0,)), ((), ())),
                                      preferred_element_type=jnp.float32)
        state_ref[h] = s_h * es_last[0:1, h:h + 1] + contrib
        ys.append(y_h)

    y = jnp.concatenate(ys, axis=1) + x_full * d_row_ref[...]

    yg = y * (z * jax.nn.sigmoid(z))
    ms = jnp.mean(yg * yg, axis=-1, keepdims=True)
    yn = yg * jax.lax.rsqrt(ms + RMS_EPS) * nw_ref[...]

    o_ref[0] = jnp.dot(yn.astype(jnp.bfloat16), wo_ref[...],
                       preferred_element_type=jnp.float32)


def kernel(u, in_proj_w, conv_w, conv_b, dt_bias, A_log, D_param, norm_w, out_proj_w):
    f32 = jnp.float32
    bf16 = jnp.bfloat16
    wz_t = in_proj_w[:D_INNER].T.astype(bf16)
    wxbc_t = in_proj_w[D_INNER:D_INNER + CONV_DIM].T.astype(bf16)
    wdt_o = in_proj_w[D_INNER + CONV_DIM:]
    wdt_t = wdt_o.T
    cw4 = conv_w[:, 0, :].T
    cb = conv_b.reshape(1, CONV_DIM)
    dtb_row = dt_bias.reshape(1, NHEADS)
    dtb_col = dt_bias.reshape(NHEADS, 1)
    a_row = (-jnp.exp(A_log)).reshape(1, NHEADS)
    a_col = a_row.reshape(NHEADS, 1)
    d_row = jnp.repeat(D_param, HEADDIM).reshape(1, D_INNER)
    nw = norm_w.reshape(1, D_INNER)
    wo_t = out_proj_w.T.astype(bf16)

    nt = SEQLEN // Q
    grid = (BATCH, nt)
    full = lambda shp: pl.BlockSpec(shp, lambda b, t: (0,) * len(shp))
    out = pl.pallas_call(
        _mamba_kernel,
        grid=grid,
        in_specs=[
            pl.BlockSpec((1, Q, D_MODEL), lambda b, t: (b, t, 0)),
            full((D_MODEL, D_INNER)),
            full((D_MODEL, CONV_DIM)),
            full((D_MODEL, NHEADS)),
            full((NHEADS, D_MODEL)),
            full((D_CONV, CONV_DIM)),
            full((1, CONV_DIM)),
            full((1, NHEADS)),
            full((NHEADS, 1)),
            full((1, NHEADS)),
            full((NHEADS, 1)),
            full((1, D_INNER)),
            full((1, D_INNER)),
            full((D_INNER, D_MODEL)),
        ],
        out_specs=pl.BlockSpec((1, Q, D_MODEL), lambda b, t: (b, t, 0)),
        out_shape=jax.ShapeDtypeStruct((BATCH, SEQLEN, D_MODEL), f32),
        scratch_shapes=[
            pltpu.VMEM((8, CONV_DIM), f32),
            pltpu.VMEM((NHEADS, D_STATE, HEADDIM), f32),
        ],
        compiler_params=pltpu.CompilerParams(
            dimension_semantics=("parallel", "arbitrary"),
            vmem_limit_bytes=100 * 1024 * 1024,
        ),
    )(u.astype(f32), wz_t, wxbc_t, wdt_t, wdt_o, cw4, cb, dtb_row, dtb_col,
      a_row, a_col, d_row, nw, wo_t)
    return out


# speedup vs baseline: 72.6595x; 1.1284x over previous
import jax
import jax.numpy as jnp
from jax.experimental import pallas as pl
from jax.experimental.pallas import tpu as pltpu

D_MODEL = 1024
D_STATE = 64
D_CONV = 4
HEADDIM = 128
NHEADS = 16
D_INNER = 2048
CONV_DIM = 2176
BATCH = 2
SEQLEN = 2048
Q = 256
RMS_EPS = 1e-5
NEG_BIG = -1e30


def _softplus(x):
    return jnp.maximum(x, 0.0) + jnp.log1p(jnp.exp(-jnp.abs(x)))


def _mamba_kernel(u_ref, wz_ref, wdt_ref, wdt_o_ref, cw_ref, cb_ref,
                  dtb_row_ref, dtb_col_ref, a_row_ref, a_col_ref, d_row_ref,
                  nw_ref, wo_ref, o_ref, xbuf_ref, state_ref):
    t = pl.program_id(1)

    @pl.when(t == 0)
    def _():
        xbuf_ref[0:D_CONV - 1] = jnp.zeros((D_CONV - 1, CONV_DIM), jnp.float32)
        state_ref[...] = jnp.zeros_like(state_ref)

    uv = u_ref[0]
    uvb = uv.astype(jnp.bfloat16)

    cdims = (((1,), (1,)), ((), ()))
    z = jax.lax.dot_general(uvb, wz_ref[0:D_INNER], cdims,
                            preferred_element_type=jnp.float32)
    xbc = jax.lax.dot_general(uvb, wz_ref[D_INNER:D_INNER + CONV_DIM], cdims,
                              preferred_element_type=jnp.float32)
    dtr = jnp.dot(uv, wdt_ref[...], preferred_element_type=jnp.float32)
    dtr_t = jax.lax.dot_general(wdt_o_ref[...], uv, (((1,), (1,)), ((), ())),
                                preferred_element_type=jnp.float32)

    xbuf_ref[D_CONV - 1:D_CONV - 1 + Q] = xbc
    conv = (cw_ref[0:1] * xbuf_ref[0:Q] + cw_ref[1:2] * xbuf_ref[1:1 + Q]
            + cw_ref[2:3] * xbuf_ref[2:2 + Q] + cw_ref[3:4] * xbuf_ref[3:3 + Q]
            + cb_ref[...])
    xbuf_ref[0:D_CONV - 1] = xbuf_ref[Q:Q + D_CONV - 1]
    xbc_a = conv * jax.nn.sigmoid(conv)

    x_full = xbc_a[:, :D_INNER]
    bmat = xbc_a[:, D_INNER:D_INNER + D_STATE]
    cmat = xbc_a[:, D_INNER + D_STATE:]

    dt = _softplus(dtr + dtb_row_ref[...])
    dt_t = _softplus(dtr_t + dtb_col_ref[...])
    a = dt * a_row_ref[...]
    a_t = dt_t * a_col_ref[...]

    ri = jax.lax.broadcasted_iota(jnp.int32, (Q, Q), 0)
    ci = jax.lax.broadcasted_iota(jnp.int32, (Q, Q), 1)
    causal = ci <= ri
    trif = jnp.where(causal, 1.0, 0.0)
    negmask = jnp.where(causal, 0.0, NEG_BIG)

    s = jnp.dot(trif, a, preferred_element_type=jnp.float32)
    s_t = jax.lax.dot_general(a_t, trif, (((1,), (1,)), ((), ())),
                              preferred_element_type=jnp.float32)

    es16 = jnp.exp(s)
    es_last = es16[Q - 1:Q, :]
    wd16 = jnp.exp(s[Q - 1:Q, :] - s) * dt

    he = jax.lax.broadcasted_iota(jnp.int32, (16, D_INNER), 0)
    le = jax.lax.broadcasted_iota(jnp.int32, (16, D_INNER), 1)
    expand = jnp.where(he == (le >> 7), 1.0, 0.0).astype(jnp.bfloat16)
    dt_exp = jnp.dot(dt.astype(jnp.bfloat16), expand,
                     preferred_element_type=jnp.float32).astype(jnp.bfloat16)
    wd_exp = jnp.dot(wd16.astype(jnp.bfloat16), expand,
                     preferred_element_type=jnp.float32).astype(jnp.bfloat16)
    es_exp = jnp.dot(es16.astype(jnp.bfloat16), expand,
                     preferred_element_type=jnp.float32)

    x_b = x_full.astype(jnp.bfloat16)
    dtx_b = x_b * dt_exp
    xw_b = x_b * wd_exp

    bmat_b = bmat.astype(jnp.bfloat16)
    cmat_b = cmat.astype(jnp.bfloat16)
    g = jax.lax.dot_general(cmat_b, bmat_b, (((1,), (1,)), ((), ())),
                            preferred_element_type=jnp.float32)

    hi16 = jax.lax.broadcasted_iota(jnp.int32, (16, Q), 0)
    ys = []
    for h in range(NHEADS):
        lo, hihd = h * HEADDIM, (h + 1) * HEADDIM
        w1h = jnp.where(hi16 == h, 1.0, 0.0)
        col_s = jnp.dot(s, w1h, preferred_element_type=jnp.float32)
        row_s = s_t[h:h + 1, :]
        m = jnp.exp(col_s - row_s + negmask)
        p = (g * m).astype(jnp.bfloat16)
        s_h = state_ref[h]
        y_h = (jnp.dot(p, dtx_b[:, lo:hihd], preferred_element_type=jnp.float32)
               + es_exp[:, lo:hihd]
               * jnp.dot(cmat_b, s_h.astype(jnp.bfloat16),
                         preferred_element_type=jnp.float32))
        contrib = jax.lax.dot_general(bmat_b, xw_b[:, lo:hihd],
                                      (((0,), (0,)), ((), ())),
                                      preferred_element_type=jnp.float32)
        state_ref[h] = s_h * es_last[0:1, h:h + 1] + contrib
        ys.append(y_h)

    y = jnp.concatenate(ys, axis=1) + x_full * d_row_ref[...]

    yg = y * (z * jax.nn.sigmoid(z))
    ms = jnp.mean(yg * yg, axis=-1, keepdims=True)
    yn = yg * jax.lax.rsqrt(ms + RMS_EPS) * nw_ref[...]

    o_ref[0] = jax.lax.dot_general(yn.astype(jnp.bfloat16), wo_ref[...],
                                   cdims, preferred_element_type=jnp.float32)


def kernel(u, in_proj_w, conv_w, conv_b, dt_bias, A_log, D_param, norm_w, out_proj_w):
    f32 = jnp.float32
    bf16 = jnp.bfloat16
    wzx = in_proj_w[:D_INNER + CONV_DIM].astype(bf16)
    wdt_o = in_proj_w[D_INNER + CONV_DIM:]
    wdt_t = wdt_o.T
    cw4 = conv_w[:, 0, :].T
    cb = conv_b.reshape(1, CONV_DIM)
    dtb_row = dt_bias.reshape(1, NHEADS)
    dtb_col = dt_bias.reshape(NHEADS, 1)
    a_row = (-jnp.exp(A_log)).reshape(1, NHEADS)
    a_col = a_row.reshape(NHEADS, 1)
    d_row = jnp.repeat(D_param, HEADDIM).reshape(1, D_INNER)
    nw = norm_w.reshape(1, D_INNER)
    wo_b = out_proj_w.astype(bf16)

    nt = SEQLEN // Q
    grid = (BATCH, nt)
    full = lambda shp: pl.BlockSpec(shp, lambda b, t: (0,) * len(shp))
    out = pl.pallas_call(
        _mamba_kernel,
        grid=grid,
        in_specs=[
            pl.BlockSpec((1, Q, D_MODEL), lambda b, t: (b, t, 0)),
            full((D_INNER + CONV_DIM, D_MODEL)),
            full((D_MODEL, NHEADS)),
            full((NHEADS, D_MODEL)),
            full((D_CONV, CONV_DIM)),
            full((1, CONV_DIM)),
            full((1, NHEADS)),
            full((NHEADS, 1)),
            full((1, NHEADS)),
            full((NHEADS, 1)),
            full((1, D_INNER)),
            full((1, D_INNER)),
            full((D_MODEL, D_INNER)),
        ],
        out_specs=pl.BlockSpec((1, Q, D_MODEL), lambda b, t: (b, t, 0)),
        out_shape=jax.ShapeDtypeStruct((BATCH, SEQLEN, D_MODEL), f32),
        scratch_shapes=[
            pltpu.VMEM((Q + 8, CONV_DIM), f32),
            pltpu.VMEM((NHEADS, D_STATE, HEADDIM), f32),
        ],
        compiler_params=pltpu.CompilerParams(
            dimension_semantics=("parallel", "arbitrary"),
            vmem_limit_bytes=100 * 1024 * 1024,
        ),
    )(u.astype(f32), wzx, wdt_t, wdt_o, cw4, cb, dtb_row, dtb_col,
      a_row, a_col, d_row, nw, wo_b)
    return out
